# baseline (speedup 1.0000x reference)
import jax
import jax.numpy as jnp
from jax.experimental import pallas as pl
from jax.experimental.pallas import tpu as pltpu
from jax.experimental.pallas import tpu_sc as plsc

E = 8
D = 768
F = 3072
T = 2048
TB = 512
NT = T // TB

B = 256
NBLK = 24
NSLOT = NBLK * B
NA = 2 * T
CH = 128
W = 128
DP = D // 2


def _router_index_kernel(x_ref, wr_ref, g_ref, pos_ref, binfo_ref):
    logits = jnp.dot(x_ref[...], wr_ref[...], preferred_element_type=jnp.float32)
    col = jax.lax.broadcasted_iota(jnp.int32, (T, E), 1)
    v1 = jnp.max(logits, axis=-1, keepdims=True)
    i1 = jnp.argmax(logits, axis=-1)[:, None]
    masked = jnp.where(col == i1, -jnp.inf, logits)
    v2 = jnp.max(masked, axis=-1, keepdims=True)
    i2 = jnp.argmax(masked, axis=-1)[:, None]
    e21 = jnp.exp(v2 - v1)
    g1 = 1.0 / (1.0 + e21)
    g2 = e21 / (1.0 + e21)
    g_ref[...] = jnp.concatenate([g1, g2], axis=1)

    A1 = (col == i1).astype(jnp.float32)
    A2 = (col == i2).astype(jnp.float32)
    cnt = jnp.sum(A1, axis=0, keepdims=True) + jnp.sum(A2, axis=0, keepdims=True)
    pcntb = jnp.floor((cnt + (B - 1)) / B)
    r8 = jax.lax.broadcasted_iota(jnp.int32, (E, E), 0)
    c8 = jax.lax.broadcasted_iota(jnp.int32, (E, E), 1)
    excl = (r8 < c8).astype(jnp.float32)
    bstart = jnp.dot(pcntb, excl, preferred_element_type=jnp.float32)
    poff = bstart * B
    total_blocks = jnp.sum(pcntb)

    rl = jax.lax.broadcasted_iota(jnp.int32, (CH, CH), 0)
    cl = jax.lax.broadcasted_iota(jnp.int32, (CH, CH), 1)
    lt = (cl < rl).astype(jnp.float32)
    off = jnp.zeros((1, E), jnp.float32)
    chunks = []
    for a_mat in (A1, A2):
        for c in range(T // CH):
            ac = a_mat[c * CH:(c + 1) * CH, :]
            p = jnp.dot(lt, ac, preferred_element_type=jnp.float32) + off
            chunks.append(jnp.sum((p + poff) * ac, axis=1, keepdims=True))
            off = off + jnp.sum(ac, axis=0, keepdims=True)
    pos_ref[...] = jnp.concatenate(chunks, axis=0).astype(jnp.int32)

    jcol = jax.lax.broadcasted_iota(jnp.int32, (32, 1), 0).astype(jnp.float32)
    be = jnp.sum((jcol >= bstart).astype(jnp.int32), axis=1, keepdims=True) - 1
    be = jnp.clip(be, 0, E - 1)
    bv = (jcol < total_blocks).astype(jnp.int32)
    binfo_ref[...] = jnp.concatenate([be, bv], axis=1)


def _ffn_kernel(be_ref, bv_ref, xs_ref, w1_ref, b1_ref, w2_ref, b2_ref, o_ref):
    j = pl.program_id(0)

    @pl.when(bv_ref[j] == 1)
    def _():
        h = jnp.dot(xs_ref[...].astype(jnp.float32), w1_ref[0],
                    preferred_element_type=jnp.float32)
        h = jnp.maximum(h + b1_ref[0], 0.0)
        y = jnp.dot(h, w2_ref[0], preferred_element_type=jnp.float32) + b2_ref[0]
        o_ref[...] = y.astype(jnp.bfloat16)


def _combine_kernel(yu1_ref, yu2_ref, g_ref, o_ref):
    o_ref[...] = (g_ref[:, 0:1] * yu1_ref[...].astype(jnp.float32)
                  + g_ref[:, 1:2] * yu2_ref[...].astype(jnp.float32))


def kernel(x, W_router, W1, b1, W2, b2):
    g, pos, binfo = pl.pallas_call(
        _router_index_kernel,
        out_shape=(
            jax.ShapeDtypeStruct((T, 2), jnp.float32),
            jax.ShapeDtypeStruct((NA, 1), jnp.int32),
            jax.ShapeDtypeStruct((32, 2), jnp.int32),
        ),
        in_specs=[
            pl.BlockSpec((T, D), lambda: (0, 0)),
            pl.BlockSpec((D, E), lambda: (0, 0)),
        ],
        out_specs=(
            pl.BlockSpec((T, 2), lambda: (0, 0)),
            pl.BlockSpec((NA, 1), lambda: (0, 0)),
            pl.BlockSpec((32, 2), lambda: (0, 0)),
        ),
    )(x, W_router)

    pos_rows = pos.reshape(NA // W, W)
    mesh = plsc.VectorSubcoreMesh(core_axis_name="c", subcore_axis_name="s")
    xp = jax.lax.bitcast_convert_type(
        x.astype(jnp.bfloat16).reshape(T, DP, 2), jnp.int32)

    @pl.kernel(out_type=jax.ShapeDtypeStruct((NSLOT, DP), jnp.int32), mesh=mesh)
    def _dispatch(xb_hbm, posr_hbm, xs_hbm):
        def body(x_vmem, i_vmem):
            pltpu.sync_copy(x_vmem, xs_hbm.at[i_vmem.at[0]])

        pltpu.emit_pipeline(
            body,
            grid=(2, T // W),
            in_specs=[
                pl.BlockSpec((W, DP), lambda k, w: (w, 0)),
                pl.BlockSpec((1, W), lambda k, w: (k * (T // W) + w, 0)),
            ],
            out_specs=[],
            core_axis_name=("c", "s"),
            dimension_semantics=(pltpu.PARALLEL, pltpu.PARALLEL),
        )(xb_hbm, posr_hbm)

    xs = jax.lax.bitcast_convert_type(
        _dispatch(xp, pos_rows), jnp.bfloat16).reshape(NSLOT, D)

    b1r = b1.reshape(E, 1, F)
    b2r = b2.reshape(E, 1, D)

    ys = pl.pallas_call(
        _ffn_kernel,
        grid_spec=pltpu.PrefetchScalarGridSpec(
            num_scalar_prefetch=2,
            grid=(NBLK,),
            in_specs=[
                pl.BlockSpec((B, D), lambda j, be, bv: (j, 0)),
                pl.BlockSpec((1, D, F), lambda j, be, bv: (be[j], 0, 0)),
                pl.BlockSpec((1, 1, F), lambda j, be, bv: (be[j], 0, 0)),
                pl.BlockSpec((1, F, D), lambda j, be, bv: (be[j], 0, 0)),
                pl.BlockSpec((1, 1, D), lambda j, be, bv: (be[j], 0, 0)),
            ],
            out_specs=pl.BlockSpec((B, D), lambda j, be, bv: (j, 0)),
        ),
        out_shape=jax.ShapeDtypeStruct((NSLOT, D), jnp.bfloat16),
        compiler_params=pltpu.CompilerParams(
            dimension_semantics=("arbitrary",),
        ),
    )(binfo[:, 0], binfo[:, 1], xs, W1, b1r, W2, b2r)

    ysp = jax.lax.bitcast_convert_type(ys.reshape(NSLOT, DP, 2), jnp.int32)

    @pl.kernel(out_type=jax.ShapeDtypeStruct((NA, DP), jnp.int32), mesh=mesh)
    def _collect(ys_hbm, posr_hbm, yu_hbm):
        def body(i_vmem, o_vmem):
            pltpu.sync_copy(ys_hbm.at[i_vmem.at[0]], o_vmem)

        pltpu.emit_pipeline(
            body,
            grid=(NA // W,),
            in_specs=[pl.BlockSpec((1, W), lambda i: (i, 0))],
            out_specs=[pl.BlockSpec((W, DP), lambda i: (i, 0))],
            core_axis_name=("c", "s"),
            dimension_semantics=(pltpu.PARALLEL,),
        )(posr_hbm, yu_hbm)

    yu = jax.lax.bitcast_convert_type(
        _collect(ysp, pos_rows), jnp.bfloat16).reshape(NA, D)

    out = pl.pallas_call(
        _combine_kernel,
        grid=(NT,),
        out_shape=jax.ShapeDtypeStruct((T, D), jnp.float32),
        in_specs=[
            pl.BlockSpec((TB, D), lambda t: (t, 0)),
            pl.BlockSpec((TB, D), lambda t: (t + NT, 0)),
            pl.BlockSpec((TB, 2), lambda t: (t, 0)),
        ],
        out_specs=pl.BlockSpec((TB, D), lambda t: (t, 0)),
    )(yu, yu, g)
    return out


# speedup vs baseline: 3.0927x; 3.0927x over previous
import jax
import jax.numpy as jnp
from jax.experimental import pallas as pl
from jax.experimental.pallas import tpu as pltpu
from jax.experimental.pallas import tpu_sc as plsc

E = 8
D = 768
F = 3072
T = 2048
TB = 512
NT = T // TB

B = 256
NBLK = 24
NSLOT = NBLK * B
NA = 2 * T
CH = 128
W = 64


def _router_index_kernel(x_ref, wr_ref, g_ref, pos_ref, binfo_ref):
    logits = jnp.dot(x_ref[...], wr_ref[...], preferred_element_type=jnp.float32)
    col = jax.lax.broadcasted_iota(jnp.int32, (T, E), 1)
    v1 = jnp.max(logits, axis=-1, keepdims=True)
    i1 = jnp.argmax(logits, axis=-1)[:, None]
    masked = jnp.where(col == i1, -jnp.inf, logits)
    v2 = jnp.max(masked, axis=-1, keepdims=True)
    i2 = jnp.argmax(masked, axis=-1)[:, None]
    e21 = jnp.exp(v2 - v1)
    g1 = 1.0 / (1.0 + e21)
    g2 = e21 / (1.0 + e21)
    g_ref[...] = jnp.concatenate([g1, g2], axis=1)

    A1 = (col == i1).astype(jnp.float32)
    A2 = (col == i2).astype(jnp.float32)
    cnt = jnp.sum(A1, axis=0, keepdims=True) + jnp.sum(A2, axis=0, keepdims=True)
    pcntb = jnp.floor((cnt + (B - 1)) / B)
    r8 = jax.lax.broadcasted_iota(jnp.int32, (E, E), 0)
    c8 = jax.lax.broadcasted_iota(jnp.int32, (E, E), 1)
    excl = (r8 < c8).astype(jnp.float32)
    bstart = jnp.dot(pcntb, excl, preferred_element_type=jnp.float32)
    poff = bstart * B
    total_blocks = jnp.sum(pcntb)

    rl = jax.lax.broadcasted_iota(jnp.int32, (CH, CH), 0)
    cl = jax.lax.broadcasted_iota(jnp.int32, (CH, CH), 1)
    lt = (cl < rl).astype(jnp.float32)
    off = jnp.zeros((1, E), jnp.float32)
    chunks = []
    for a_mat in (A1, A2):
        for c in range(T // CH):
            ac = a_mat[c * CH:(c + 1) * CH, :]
            p = jnp.dot(lt, ac, preferred_element_type=jnp.float32) + off
            chunks.append(jnp.sum((p + poff) * ac, axis=1, keepdims=True))
            off = off + jnp.sum(ac, axis=0, keepdims=True)
    pos_ref[...] = jnp.concatenate(chunks, axis=0).astype(jnp.int32)

    jcol = jax.lax.broadcasted_iota(jnp.int32, (32, 1), 0).astype(jnp.float32)
    be = jnp.sum((jcol >= bstart).astype(jnp.int32), axis=1, keepdims=True) - 1
    be = jnp.clip(be, 0, E - 1)
    bv = (jcol < total_blocks).astype(jnp.int32)
    binfo_ref[...] = jnp.concatenate([be, bv], axis=1)


def _ffn_kernel(be_ref, bv_ref, xs_ref, w1_ref, b1_ref, w2_ref, b2_ref, o_ref):
    j = pl.program_id(0)

    @pl.when(bv_ref[j] == 1)
    def _():
        h = jnp.dot(xs_ref[...], w1_ref[0], preferred_element_type=jnp.float32)
        h = jnp.maximum(h + b1_ref[0], 0.0)
        o_ref[...] = jnp.dot(h, w2_ref[0], preferred_element_type=jnp.float32) + b2_ref[0]


def _combine_kernel(yu1_ref, yu2_ref, g_ref, o_ref):
    o_ref[...] = (g_ref[:, 0:1] * yu1_ref[...] + g_ref[:, 1:2] * yu2_ref[...])


def kernel(x, W_router, W1, b1, W2, b2):
    g, pos, binfo = pl.pallas_call(
        _router_index_kernel,
        out_shape=(
            jax.ShapeDtypeStruct((T, 2), jnp.float32),
            jax.ShapeDtypeStruct((NA, 1), jnp.int32),
            jax.ShapeDtypeStruct((32, 2), jnp.int32),
        ),
        in_specs=[
            pl.BlockSpec((T, D), lambda: (0, 0)),
            pl.BlockSpec((D, E), lambda: (0, 0)),
        ],
        out_specs=(
            pl.BlockSpec((T, 2), lambda: (0, 0)),
            pl.BlockSpec((NA, 1), lambda: (0, 0)),
            pl.BlockSpec((32, 2), lambda: (0, 0)),
        ),
    )(x, W_router)

    pos_rows = pos.reshape(NA // W, W)
    mesh = plsc.VectorSubcoreMesh(core_axis_name="c", subcore_axis_name="s")

    @pl.kernel(out_type=jax.ShapeDtypeStruct((NSLOT, D), jnp.float32), mesh=mesh)
    def _dispatch(xb_hbm, posr_hbm, xs_hbm):
        def body(x_vmem, i_vmem):
            pltpu.sync_copy(x_vmem, xs_hbm.at[i_vmem.at[0]])

        pltpu.emit_pipeline(
            body,
            grid=(2, T // W),
            in_specs=[
                pl.BlockSpec((W, D), lambda k, w: (w, 0)),
                pl.BlockSpec((1, W), lambda k, w: (k * (T // W) + w, 0)),
            ],
            out_specs=[],
            core_axis_name=("c", "s"),
            dimension_semantics=(pltpu.PARALLEL, pltpu.PARALLEL),
        )(xb_hbm, posr_hbm)

    xs = _dispatch(x, pos_rows)

    b1r = b1.reshape(E, 1, F)
    b2r = b2.reshape(E, 1, D)

    ys = pl.pallas_call(
        _ffn_kernel,
        grid_spec=pltpu.PrefetchScalarGridSpec(
            num_scalar_prefetch=2,
            grid=(NBLK,),
            in_specs=[
                pl.BlockSpec((B, D), lambda j, be, bv: (j, 0)),
                pl.BlockSpec((1, D, F), lambda j, be, bv: (be[j], 0, 0)),
                pl.BlockSpec((1, 1, F), lambda j, be, bv: (be[j], 0, 0)),
                pl.BlockSpec((1, F, D), lambda j, be, bv: (be[j], 0, 0)),
                pl.BlockSpec((1, 1, D), lambda j, be, bv: (be[j], 0, 0)),
            ],
            out_specs=pl.BlockSpec((B, D), lambda j, be, bv: (j, 0)),
        ),
        out_shape=jax.ShapeDtypeStruct((NSLOT, D), jnp.float32),
        compiler_params=pltpu.CompilerParams(
            dimension_semantics=("arbitrary",),
        ),
    )(binfo[:, 0], binfo[:, 1], xs, W1, b1r, W2, b2r)

    @pl.kernel(out_type=jax.ShapeDtypeStruct((NA, D), jnp.float32), mesh=mesh)
    def _collect(ys_hbm, posr_hbm, yu_hbm):
        def body(i_vmem, o_vmem):
            pltpu.sync_copy(ys_hbm.at[i_vmem.at[0]], o_vmem)

        pltpu.emit_pipeline(
            body,
            grid=(NA // W,),
            in_specs=[pl.BlockSpec((1, W), lambda i: (i, 0))],
            out_specs=[pl.BlockSpec((W, D), lambda i: (i, 0))],
            core_axis_name=("c", "s"),
            dimension_semantics=(pltpu.PARALLEL,),
        )(posr_hbm, yu_hbm)

    yu = _collect(ys, pos_rows)

    out = pl.pallas_call(
        _combine_kernel,
        grid=(NT,),
        out_shape=jax.ShapeDtypeStruct((T, D), jnp.float32),
        in_specs=[
            pl.BlockSpec((TB, D), lambda t: (t, 0)),
            pl.BlockSpec((TB, D), lambda t: (t + NT, 0)),
            pl.BlockSpec((TB, 2), lambda t: (t, 0)),
        ],
        out_specs=pl.BlockSpec((TB, D), lambda t: (t, 0)),
    )(yu, yu, g)
    return out
